# Initial kernel scaffold; baseline (speedup 1.0000x reference)
import jax
import jax.numpy as jnp
from jax.experimental import pallas as pl


def kernel(x, weights):
    raise NotImplementedError("write your pallas kernel here")


# speedup vs baseline: 3.9260x; 3.9260x over previous
import functools

import jax
import jax.numpy as jnp
from jax import lax
from jax.experimental import pallas as pl
from jax.experimental.pallas import tpu as pltpu
from jax.experimental.pallas import tpu_sc as plsc

_V = 201
_D = 64 * 64
_B = 16384


@functools.lru_cache(maxsize=None)
def _make_gather():
    info = plsc.get_sparse_core_info()
    nw = info.num_cores * info.num_subcores
    b_per_w = _B // nw
    k = 16
    nchunks = b_per_w // k
    mesh = plsc.VectorSubcoreMesh(core_axis_name="c", subcore_axis_name="s")

    @functools.partial(
        pl.kernel,
        out_type=jax.ShapeDtypeStruct((_B, _D), jnp.float32),
        mesh=mesh,
        scratch_types=[
            pltpu.VMEM((b_per_w,), jnp.int32),
            pltpu.VMEM((k, _D), jnp.float32),
            pltpu.SemaphoreType.DMA,
        ],
    )
    def gather(idx_hbm, table_hbm, out_hbm, idx_v, rows_v, sem):
        wid = lax.axis_index("s") * info.num_cores + lax.axis_index("c")
        base = wid * b_per_w
        pltpu.sync_copy(idx_hbm.at[pl.ds(base, b_per_w)], idx_v)

        def body(c, carry):
            off = pl.multiple_of(c * k, 8)
            pltpu.async_copy(
                table_hbm.at[idx_v.at[pl.ds(off, k)]], rows_v, sem
            ).wait()
            pltpu.sync_copy(rows_v, out_hbm.at[pl.ds(base + off, k)])
            return carry

        lax.fori_loop(0, nchunks, body, 0)

    return gather


def kernel(x, weights):
    table = weights.reshape(_V, _D)
    return _make_gather()(x, table)


# speedup vs baseline: 4.1037x; 1.0453x over previous
import functools

import jax
import jax.numpy as jnp
from jax import lax
from jax.experimental import pallas as pl
from jax.experimental.pallas import tpu as pltpu
from jax.experimental.pallas import tpu_sc as plsc

_V = 201
_D = 64 * 64
_B = 16384


@functools.lru_cache(maxsize=None)
def _make_gather():
    info = plsc.get_sparse_core_info()
    nw = info.num_cores * info.num_subcores
    b_per_w = _B // nw
    k = 8
    nchunks = b_per_w // k
    mesh = plsc.VectorSubcoreMesh(core_axis_name="c", subcore_axis_name="s")

    @functools.partial(
        pl.kernel,
        out_type=jax.ShapeDtypeStruct((_B, _D), jnp.float32),
        mesh=mesh,
        scratch_types=[
            pltpu.VMEM((b_per_w,), jnp.int32),
            pltpu.VMEM((k, _D), jnp.float32),
            pltpu.VMEM((k, _D), jnp.float32),
            pltpu.SemaphoreType.DMA,
            pltpu.SemaphoreType.DMA,
        ],
    )
    def gather(idx_hbm, table_hbm, out_hbm, idx_v, rows0, rows1, sem0, sem1):
        wid = lax.axis_index("s") * info.num_cores + lax.axis_index("c")
        base = wid * b_per_w
        pltpu.sync_copy(idx_hbm.at[pl.ds(base, b_per_w)], idx_v)
        bufs = (rows0, rows1)
        sems = (sem0, sem1)

        def issue(c, buf, sem):
            off = pl.multiple_of(c * k, 8)
            pltpu.async_copy(table_hbm.at[idx_v.at[pl.ds(off, k)]], buf, sem)

        def drain(c, buf, sem):
            off = pl.multiple_of(c * k, 8)
            pltpu.make_async_copy(
                table_hbm.at[idx_v.at[pl.ds(off, k)]], buf, sem
            ).wait()

        issue(0, bufs[0], sems[0])

        def body(p, carry):
            for b in range(2):
                c = p * 2 + b
                drain(c, bufs[b], sems[b])
                pl.when(c + 1 < nchunks)(
                    lambda: issue(c + 1, bufs[1 - b], sems[1 - b])
                )
                pltpu.sync_copy(bufs[b], out_hbm.at[pl.ds(base + c * k, k)])
            return carry

        lax.fori_loop(0, nchunks // 2, body, 0)

    return gather


def kernel(x, weights):
    table = weights.reshape(_V, _D)
    return _make_gather()(x, table)
